# Initial kernel scaffold; baseline (speedup 1.0000x reference)
import jax
import jax.numpy as jnp
from jax.experimental import pallas as pl


def kernel(x, edge_index, W0, b0, W1, b1):
    raise NotImplementedError("write your pallas kernel here")


# speedup vs baseline: 15.1068x; 15.1068x over previous
import functools

import jax
import jax.numpy as jnp
from jax import lax
from jax.experimental import pallas as pl
from jax.experimental.pallas import tpu as pltpu
from jax.experimental.pallas import tpu_sc as plsc

N = 10000
E = 320000
D = 128
N_PAD = 10240
NC = 2
NS = 16
NW = NC * NS
EPT = E // NW
CHUNK = 80
NCHUNK = EPT // CHUNK
RPT = N_PAD // NS
ZR = 40
BM = 1000

_mesh = plsc.VectorSubcoreMesh(core_axis_name="c", subcore_axis_name="s")


@functools.partial(
    pl.kernel,
    mesh=_mesh,
    out_type=jax.ShapeDtypeStruct((NC, 2, N_PAD), jnp.float32),
    scratch_types=[
        pltpu.VMEM((NCHUNK, CHUNK), jnp.int32),
        pltpu.VMEM((NCHUNK, CHUNK), jnp.int32),
        pltpu.VMEM((CHUNK,), jnp.float32),
        pltpu.VMEM((RPT,), jnp.float32),
        pltpu.VMEM_SHARED((N_PAD,), jnp.float32),
        pltpu.VMEM_SHARED((N_PAD,), jnp.float32),
    ],
)
def _deg_kernel(src_hbm, dst_hbm, ones_hbm, zeros_hbm, out_hbm,
                src_v, dst_v, ones_v, buf_v, dego_sh, degi_sh):
    cid = lax.axis_index("c")
    sid = lax.axis_index("s")
    wid = sid * NC + cid
    pltpu.sync_copy(src_hbm.at[wid], src_v)
    pltpu.sync_copy(dst_hbm.at[wid], dst_v)
    pltpu.sync_copy(ones_hbm, ones_v)
    pltpu.sync_copy(zeros_hbm, buf_v)
    pltpu.sync_copy(buf_v, dego_sh.at[pl.ds(sid * RPT, RPT)])
    pltpu.sync_copy(buf_v, degi_sh.at[pl.ds(sid * RPT, RPT)])
    plsc.subcore_barrier()

    def body(j, carry):
        pltpu.sync_copy(ones_v, dego_sh.at[src_v.at[j]], add=True)
        pltpu.sync_copy(ones_v, degi_sh.at[dst_v.at[j]], add=True)
        return carry

    lax.fori_loop(0, NCHUNK, body, 0)
    plsc.subcore_barrier()
    pltpu.sync_copy(dego_sh.at[pl.ds(sid * RPT, RPT)], buf_v)
    pltpu.sync_copy(buf_v, out_hbm.at[cid, 0, pl.ds(sid * RPT, RPT)])
    pltpu.sync_copy(degi_sh.at[pl.ds(sid * RPT, RPT)], buf_v)
    pltpu.sync_copy(buf_v, out_hbm.at[cid, 1, pl.ds(sid * RPT, RPT)])


@functools.partial(
    pl.kernel,
    mesh=_mesh,
    out_type=jax.ShapeDtypeStruct((NC, N_PAD, D), jnp.float32),
    scratch_types=[
        pltpu.VMEM((NCHUNK, CHUNK), jnp.int32),
        pltpu.VMEM((NCHUNK, CHUNK), jnp.int32),
        pltpu.VMEM((CHUNK, D), jnp.float32),
        pltpu.VMEM((ZR, D), jnp.float32),
        pltpu.VMEM_SHARED((N_PAD, D), jnp.float32),
        pltpu.SemaphoreType.DMA,
    ],
)
def _edge_kernel(h_hbm, src_hbm, dst_hbm, zrows_hbm, out_hbm,
                 src_v, dst_v, rows_v, zbuf_v, acc_sh, sem):
    cid = lax.axis_index("c")
    sid = lax.axis_index("s")
    wid = sid * NC + cid
    pltpu.sync_copy(src_hbm.at[wid], src_v)
    pltpu.sync_copy(dst_hbm.at[wid], dst_v)
    pltpu.sync_copy(zrows_hbm, zbuf_v)
    for t in range(RPT // ZR):
        pltpu.sync_copy(zbuf_v, acc_sh.at[pl.ds(sid * RPT + t * ZR, ZR)])
    plsc.subcore_barrier()

    def body(j, carry):
        pltpu.async_copy(h_hbm.at[src_v.at[j]], rows_v, sem).wait()
        pltpu.sync_copy(rows_v, acc_sh.at[dst_v.at[j]], add=True)
        return carry

    lax.fori_loop(0, NCHUNK, body, 0)
    plsc.subcore_barrier()
    for t in range(RPT // ZR):
        pltpu.sync_copy(acc_sh.at[pl.ds(sid * RPT + t * ZR, ZR)], zbuf_v)
        pltpu.sync_copy(zbuf_v, out_hbm.at[cid, pl.ds(sid * RPT + t * ZR, ZR)])


def _norm_body(p_ref, out_ref):
    deg = p_ref[0] + p_ref[1] + 1.0
    out_ref[...] = lax.rsqrt(deg)


def _mm_scale_body(x_ref, w_ref, s_ref, o_ref):
    h = jnp.dot(x_ref[...], w_ref[...], preferred_element_type=jnp.float32)
    o_ref[...] = h * s_ref[...]


def _combine_mm_body(p_ref, hp_ref, ni_ref, b_ref, w_ref, no_ref, o_ref):
    agg = p_ref[0] + p_ref[1] + hp_ref[...]
    h = jnp.maximum(agg * ni_ref[...] + b_ref[...], 0.0)
    o_ref[...] = jnp.dot(h, w_ref[...], preferred_element_type=jnp.float32) * no_ref[...]


def _combine_final_body(p_ref, hp_ref, ni_ref, b_ref, o_ref):
    agg = p_ref[0] + p_ref[1] + hp_ref[...]
    o_ref[...] = agg * ni_ref[...] + b_ref[...]


def _norms(deg_p):
    return pl.pallas_call(
        _norm_body,
        out_shape=jax.ShapeDtypeStruct((2, N_PAD), jnp.float32),
    )(deg_p)


def _mm_scale(xv, W, s_col):
    return pl.pallas_call(
        _mm_scale_body,
        grid=(N // BM,),
        in_specs=[
            pl.BlockSpec((BM, D), lambda i: (i, 0)),
            pl.BlockSpec((D, D), lambda i: (0, 0)),
            pl.BlockSpec((BM, 1), lambda i: (i, 0)),
        ],
        out_specs=pl.BlockSpec((BM, D), lambda i: (i, 0)),
        out_shape=jax.ShapeDtypeStruct((N, D), jnp.float32),
    )(xv, W, s_col)


def _combine_mm(part, hp, ni_col, b_row, W, no_col):
    return pl.pallas_call(
        _combine_mm_body,
        grid=(N // BM,),
        in_specs=[
            pl.BlockSpec((NC, BM, D), lambda i: (0, i, 0)),
            pl.BlockSpec((BM, D), lambda i: (i, 0)),
            pl.BlockSpec((BM, 1), lambda i: (i, 0)),
            pl.BlockSpec((1, D), lambda i: (0, 0)),
            pl.BlockSpec((D, D), lambda i: (0, 0)),
            pl.BlockSpec((BM, 1), lambda i: (i, 0)),
        ],
        out_specs=pl.BlockSpec((BM, D), lambda i: (i, 0)),
        out_shape=jax.ShapeDtypeStruct((N, D), jnp.float32),
    )(part, hp, ni_col, b_row, W, no_col)


def _combine_final(part, hp, ni_col, b_row):
    return pl.pallas_call(
        _combine_final_body,
        grid=(N // BM,),
        in_specs=[
            pl.BlockSpec((NC, BM, D), lambda i: (0, i, 0)),
            pl.BlockSpec((BM, D), lambda i: (i, 0)),
            pl.BlockSpec((BM, 1), lambda i: (i, 0)),
            pl.BlockSpec((1, D), lambda i: (0, 0)),
        ],
        out_specs=pl.BlockSpec((BM, D), lambda i: (i, 0)),
        out_shape=jax.ShapeDtypeStruct((N, D), jnp.float32),
    )(part, hp, ni_col, b_row)


def kernel(x, edge_index, W0, b0, W1, b1):
    src = edge_index[0].reshape(NW, NCHUNK, CHUNK)
    dst = edge_index[1].reshape(NW, NCHUNK, CHUNK)
    ones_c = jnp.ones((CHUNK,), jnp.float32)
    zeros_r = jnp.zeros((RPT,), jnp.float32)
    zrows = jnp.zeros((ZR, D), jnp.float32)

    deg_p = _deg_kernel(src, dst, ones_c, zeros_r)
    norms = _norms(deg_p)
    no_col = norms[0, :N].reshape(N, 1)
    ni_col = norms[1, :N].reshape(N, 1)

    h0p = _mm_scale(x, W0, no_col)
    part0 = _edge_kernel(h0p, src, dst, zrows)
    h1p = _combine_mm(part0, h0p, ni_col, b0.reshape(1, D), W1, no_col)
    part1 = _edge_kernel(h1p, src, dst, zrows)
    return _combine_final(part1, h1p, ni_col, b1.reshape(1, D))
